# baseline (speedup 1.0000x reference)
import functools

import jax
import jax.numpy as jnp
from jax import lax
from jax.experimental import pallas as pl
from jax.experimental.pallas import tpu as pltpu
from jax.experimental.pallas import tpu_sc as plsc

VOCAB = 1000000
EMB_DIM = 16
DENSE_DIM = 13
N_SPARSE = 26
BATCH = 16384
SPARSE_FLAT = N_SPARSE * EMB_DIM

NC = 2
NS = 16
NW = NC * NS
TOT = BATCH * N_SPARSE
IDX_COLS = 128
IDX_ROWS = TOT // IDX_COLS
ROWS_PER_W = IDX_ROWS // NW
CHUNK = 13
NCHUNK = ROWS_PER_W // CHUNK
CHUNK_IDX = CHUNK * IDX_COLS


NCB = VOCAB // 128
TAIL = VOCAB - NCB * 128
CB_EXTRA = NCB - NW * (NCB // NW)


def _detile_consts():
    iota = lax.iota(jnp.int32, 16)
    consts = []
    for s in range(16):
        jv = jnp.bitwise_and(iota + s, 15)
        base = jv * 16 + iota
        consts.append((jv, jnp.right_shift(base, 7),
                       jnp.bitwise_and(base, 127)))
    return iota, consts


def _detile_cb(embT_hbm, out_hbm, tbuf, stage, sem_i, sem_s, t, wid, consts,
               fire_only):
    cb = wid + NW * t
    if fire_only:
        pltpu.async_copy(
            embT_hbm.at[pl.ds(0, 16),
                        pl.ds(pl.multiple_of(cb * 128, 128), 128)],
            tbuf, sem_i)
        return
    pltpu.make_async_copy(
        embT_hbm.at[pl.ds(0, 16), pl.ds(0, 128)], tbuf, sem_i).wait()
    iota, cs = consts
    for g in range(8):
        for s in range(16):
            jv, rows0, cols = cs[s]
            vals = plsc.load_gather(tbuf, [iota, jv + g * 16])
            plsc.store_scatter(stage, [rows0 + 2 * g, cols], vals)
    pltpu.async_copy(
        stage, out_hbm.at[pl.ds(pl.multiple_of(cb * 16, 16), 16)], sem_s)


def _detile_body(embT_hbm, tail_hbm, out_hbm,
                 tbuf0, tbuf1, stage0, stage1,
                 sem_i0, sem_i1, sem_s0, sem_s1):
    wid = lax.axis_index("s") * NC + lax.axis_index("c")
    nt = NCB // NW + jnp.where(wid < CB_EXTRA, 1, 0)
    tbufs = (tbuf0, tbuf1)
    stages = (stage0, stage1)
    sems_i = (sem_i0, sem_i1)
    sems_s = (sem_s0, sem_s1)
    consts = _detile_consts()

    def fire(t, p):
        @pl.when(t < nt)
        def _():
            _detile_cb(embT_hbm, out_hbm, tbufs[p], stages[p],
                       sems_i[p], sems_s[p], t, wid, consts, fire_only=True)

    def wait_stage(p):
        pltpu.make_async_copy(
            stages[p], out_hbm.at[pl.ds(0, 16)], sems_s[p]).wait()

    def process(t, p):
        @pl.when(t < nt)
        def _():
            _detile_cb(embT_hbm, out_hbm, tbufs[p], stages[p],
                       sems_i[p], sems_s[p], t, wid, consts, fire_only=False)

    fire(0, 0)
    fire(1, 1)

    def step(u, _):
        t0 = 2 * u
        t1 = 2 * u + 1

        @pl.when(jnp.logical_and(u >= 1, t0 < nt))
        def _():
            wait_stage(0)
        process(t0, 0)
        fire(t0 + 2, 0)

        @pl.when(jnp.logical_and(u >= 1, t1 < nt))
        def _():
            wait_stage(1)
        process(t1, 1)
        fire(t1 + 2, 1)
        return 0

    nu = (NCB // NW + 2) // 2
    lax.fori_loop(0, nu, step, 0)
    wait_stage(0)
    wait_stage(1)

    @pl.when(wid == 0)
    def _():
        pltpu.sync_copy(tail_hbm, stage0.at[pl.ds(0, 8)])
        pltpu.sync_copy(stage0.at[pl.ds(0, 8)],
                        out_hbm.at[pl.ds(NCB * 16, 8)])


@functools.cache
def _sc_detile():
    return pl.kernel(
        _detile_body,
        out_type=jax.ShapeDtypeStruct((VOCAB * EMB_DIM // 128, 128),
                                      jnp.float32),
        mesh=plsc.VectorSubcoreMesh(
            core_axis_name="c", subcore_axis_name="s",
            num_cores=NC, num_subcores=NS),
        scratch_types=[
            pltpu.VMEM((16, 128), jnp.float32),
            pltpu.VMEM((16, 128), jnp.float32),
            pltpu.VMEM((16, 128), jnp.float32),
            pltpu.VMEM((16, 128), jnp.float32),
            pltpu.SemaphoreType.DMA,
            pltpu.SemaphoreType.DMA,
            pltpu.SemaphoreType.DMA,
            pltpu.SemaphoreType.DMA,
        ],
        compiler_params=pltpu.CompilerParams(
            use_tc_tiling_on_sc=True, needs_layout_passes=False),
    )


def _sc_body(idx_hbm, idxhi_hbm, emb_hbm, one16_hbm, feat_hbm, ones_hbm,
             idx_v, idxhi_v, rows_v, onesrows_v, ones_buf, sem_e, sem_o):
    wid = lax.axis_index("s") * NC + lax.axis_index("c")
    row0 = wid * ROWS_PER_W
    emb2d = emb_hbm
    one2d = one16_hbm
    pltpu.sync_copy(idx_hbm.at[pl.ds(row0, ROWS_PER_W)], idx_v)
    pltpu.sync_copy(idxhi_hbm.at[pl.ds(row0, ROWS_PER_W)], idxhi_v)
    for c in range(NCHUNK):
        handles = []
        for j in range(CHUNK):
            r = c * CHUNK + j
            handles.append(pltpu.async_copy(
                emb2d.at[idx_v.at[r]],
                rows_v.at[pl.ds(j * IDX_COLS, IDX_COLS)], sem_e))
            handles.append(pltpu.async_copy(
                one2d.at[idxhi_v.at[r]],
                onesrows_v.at[pl.ds(j * IDX_COLS, IDX_COLS)], sem_o))
        for h in handles:
            h.wait()
        base = (row0 + c * CHUNK) * IDX_COLS
        pltpu.sync_copy(rows_v, feat_hbm.at[pl.ds(base, CHUNK_IDX)])

        def lane_select(j, _):
            row = c * CHUNK + j
            for k in range(IDX_COLS // 16):
                iv = idx_v[row, pl.ds(k * 16, 16)]
                lane = jnp.bitwise_and(iv, 15)
                off = j * IDX_COLS + k * 16
                rid = lax.iota(jnp.int32, 16) + off
                ones_buf[pl.ds(off, 16)] = plsc.load_gather(
                    onesrows_v, [rid, lane])
            return 0

        lax.fori_loop(0, CHUNK, lane_select, 0)
        pltpu.sync_copy(ones_buf, ones_hbm.at[pl.ds(base, CHUNK_IDX)])


@functools.cache
def _sc_gather():
    return pl.kernel(
        _sc_body,
        out_type=(
            jax.ShapeDtypeStruct((TOT, EMB_DIM), jnp.float32),
            jax.ShapeDtypeStruct((TOT,), jnp.float32),
        ),
        mesh=plsc.VectorSubcoreMesh(
            core_axis_name="c", subcore_axis_name="s",
            num_cores=NC, num_subcores=NS),
        scratch_types=[
            pltpu.VMEM((ROWS_PER_W, IDX_COLS), jnp.int32),
            pltpu.VMEM((ROWS_PER_W, IDX_COLS), jnp.int32),
            pltpu.VMEM((CHUNK_IDX, EMB_DIM), jnp.float32),
            pltpu.VMEM((CHUNK_IDX, EMB_DIM), jnp.float32),
            pltpu.VMEM((CHUNK_IDX,), jnp.float32),
            pltpu.SemaphoreType.DMA,
            pltpu.SemaphoreType.DMA,
        ],
        compiler_params=pltpu.CompilerParams(
            use_tc_tiling_on_sc=False, needs_layout_passes=False),
    )


R = 256
GRID = BATCH // R


def _tc_body(feat_ref, ones_ref, d_ref, dw_ref, w1_ref, S_ref, M_ref,
             W0a_ref, b0_ref, W1_ref, b1_ref, W2_ref, b2_ref, W3_ref, b3_ref,
             out_ref):
    x = feat_ref[...]
    d = d_ref[...]
    dw = dw_ref[...]
    S = S_ref[...]

    dot = functools.partial(jnp.dot, preferred_element_type=jnp.float32)

    y1 = jnp.sum(ones_ref[...], axis=1, keepdims=True) + dot(d, w1_ref[...])

    s = dot(x, S) + dot(d, dw)
    sq = dot(x * x, S) + dot(d * d, dw * dw)
    y2 = 0.5 * jnp.sum(s * s - sq, axis=1, keepdims=True)

    h = jnp.maximum(dot(x, W0a_ref[...]) + dot(d, M_ref[...]) + b0_ref[...], 0.0)
    h = jnp.maximum(dot(h, W1_ref[...]) + b1_ref[...], 0.0)
    h = jnp.maximum(dot(h, W2_ref[...]) + b2_ref[...], 0.0)
    yd = dot(h, W3_ref[...]) + b3_ref[...]

    out_ref[...] = jax.nn.sigmoid(y1 + y2 + yd)


def _tc_forward(feat, ones, dense, dw, w_one, S, M,
                W0a, b0, W1, b1, W2, b2, W3, b3):
    full = lambda shape: pl.BlockSpec(shape, lambda i: (0, 0))
    return pl.pallas_call(
        _tc_body,
        grid=(GRID,),
        in_specs=[
            pl.BlockSpec((R, SPARSE_FLAT), lambda i: (i, 0)),
            pl.BlockSpec((R, N_SPARSE), lambda i: (i, 0)),
            pl.BlockSpec((R, DENSE_DIM), lambda i: (i, 0)),
            full((DENSE_DIM, EMB_DIM)),
            full((DENSE_DIM, 1)),
            full((SPARSE_FLAT, EMB_DIM)),
            full((DENSE_DIM, 512)),
            full((SPARSE_FLAT, 512)),
            full((1, 512)),
            full((512, 256)),
            full((1, 256)),
            full((256, 128)),
            full((1, 128)),
            full((128, 1)),
            full((1, 1)),
        ],
        out_specs=pl.BlockSpec((R, 1), lambda i: (i, 0)),
        out_shape=jax.ShapeDtypeStruct((BATCH, 1), jnp.float32),
    )(feat, ones, dense, dw, w_one, S, M, W0a, b0, W1, b1, W2, b2, W3, b3)


def kernel(sparse_inputs, dense_inputs, emb_one, emb, dense_w_one, dense_w,
           W0, b0, W1, b1, W2, b2, W3, b3):
    idx = sparse_inputs.astype(jnp.int32).reshape(IDX_ROWS, IDX_COLS)
    tail128 = emb[NCB * 128:].reshape(TAIL * EMB_DIM // 128, 128)
    emb_lin = _sc_detile()(emb.T, tail128).reshape(VOCAB, EMB_DIM)
    feat_flat, ones_flat = _sc_gather()(
        idx, jnp.right_shift(idx, 4), emb_lin,
        emb_one.reshape(VOCAB // EMB_DIM, EMB_DIM))
    feat = feat_flat.reshape(BATCH, SPARSE_FLAT)
    ones = ones_flat.reshape(BATCH, N_SPARSE)

    dw = dense_w[0]
    W0a = W0[:SPARSE_FLAT]
    W0b = W0[SPARSE_FLAT:]
    M = jnp.einsum('dk,dkj->dj', dw, W0b.reshape(DENSE_DIM, EMB_DIM, 512))
    S = jnp.tile(jnp.eye(EMB_DIM, dtype=jnp.float32), (N_SPARSE, 1))

    return _tc_forward(
        feat, ones, dense_inputs, dw, dense_w_one.reshape(DENSE_DIM, 1), S, M,
        W0a, b0.reshape(1, -1), W1, b1.reshape(1, -1),
        W2, b2.reshape(1, -1), W3, b3.reshape(1, 1))


# speedup vs baseline: 1.3639x; 1.3639x over previous
import functools

import jax
import jax.numpy as jnp
from jax import lax
from jax.experimental import pallas as pl
from jax.experimental.pallas import tpu as pltpu
from jax.experimental.pallas import tpu_sc as plsc

VOCAB = 1000000
EMB_DIM = 16
DENSE_DIM = 13
N_SPARSE = 26
BATCH = 16384
SPARSE_FLAT = N_SPARSE * EMB_DIM

NC = 2
NS = 16
NW = NC * NS
TOT = BATCH * N_SPARSE
IDX_COLS = 128
IDX_ROWS = TOT // IDX_COLS
ROWS_PER_W = IDX_ROWS // NW
CHUNK = 13
NCHUNK = ROWS_PER_W // CHUNK
CHUNK_IDX = CHUNK * IDX_COLS


NCB = VOCAB // 128
TAIL = VOCAB - NCB * 128
CB_EXTRA = NCB - NW * (NCB // NW)


def _detile_consts():
    iota = lax.iota(jnp.int32, 16)
    consts = []
    for s in range(16):
        jv = jnp.bitwise_and(iota + s, 15)
        base = jv * 16 + iota
        consts.append((jv, jnp.right_shift(base, 7),
                       jnp.bitwise_and(base, 127)))
    return iota, consts


def _detile_cb(embT_hbm, out_hbm, tbuf, stage, sem_i, sem_s, t, wid, consts,
               fire_only):
    cb = wid + NW * t
    if fire_only:
        pltpu.async_copy(
            embT_hbm.at[pl.ds(0, 16),
                        pl.ds(pl.multiple_of(cb * 128, 128), 128)],
            tbuf, sem_i)
        return
    pltpu.make_async_copy(
        embT_hbm.at[pl.ds(0, 16), pl.ds(0, 128)], tbuf, sem_i).wait()
    iota, _ = consts

    @functools.partial(plsc.parallel_loop, 0, 128, unroll=8)
    def _(i):
        s = jnp.bitwise_and(i, 15)
        g16 = jnp.bitwise_and(i, 112)
        col = jnp.bitwise_and(iota + s, 15) + g16
        a = col * 16 + iota
        vals = plsc.load_gather(tbuf, [iota, col])
        plsc.store_scatter(
            stage, [jnp.right_shift(a, 7), jnp.bitwise_and(a, 127)], vals)
    pltpu.async_copy(
        stage, out_hbm.at[pl.ds(pl.multiple_of(cb * 16, 16), 16)], sem_s)


def _detile_body(embT_hbm, tail_hbm, out_hbm,
                 tbuf0, tbuf1, stage0, stage1,
                 sem_i0, sem_i1, sem_s0, sem_s1):
    wid = lax.axis_index("s") * NC + lax.axis_index("c")
    nt = NCB // NW + jnp.where(wid < CB_EXTRA, 1, 0)
    tbufs = (tbuf0, tbuf1)
    stages = (stage0, stage1)
    sems_i = (sem_i0, sem_i1)
    sems_s = (sem_s0, sem_s1)
    consts = _detile_consts()

    def fire(t, p):
        @pl.when(t < nt)
        def _():
            _detile_cb(embT_hbm, out_hbm, tbufs[p], stages[p],
                       sems_i[p], sems_s[p], t, wid, consts, fire_only=True)

    def wait_stage(p):
        pltpu.make_async_copy(
            stages[p], out_hbm.at[pl.ds(0, 16)], sems_s[p]).wait()

    def process(t, p):
        @pl.when(t < nt)
        def _():
            _detile_cb(embT_hbm, out_hbm, tbufs[p], stages[p],
                       sems_i[p], sems_s[p], t, wid, consts, fire_only=False)

    fire(0, 0)
    fire(1, 1)

    def step(u, _):
        t0 = 2 * u
        t1 = 2 * u + 1

        @pl.when(jnp.logical_and(u >= 1, t0 < nt))
        def _():
            wait_stage(0)
        process(t0, 0)
        fire(t0 + 2, 0)

        @pl.when(jnp.logical_and(u >= 1, t1 < nt))
        def _():
            wait_stage(1)
        process(t1, 1)
        fire(t1 + 2, 1)
        return 0

    nu = (NCB // NW + 2) // 2
    lax.fori_loop(0, nu, step, 0)
    wait_stage(0)
    wait_stage(1)

    @pl.when(wid == 0)
    def _():
        pltpu.sync_copy(tail_hbm, stage0.at[pl.ds(0, 8)])
        pltpu.sync_copy(stage0.at[pl.ds(0, 8)],
                        out_hbm.at[pl.ds(NCB * 16, 8)])


@functools.cache
def _sc_detile():
    return pl.kernel(
        _detile_body,
        out_type=jax.ShapeDtypeStruct((VOCAB * EMB_DIM // 128, 128),
                                      jnp.float32),
        mesh=plsc.VectorSubcoreMesh(
            core_axis_name="c", subcore_axis_name="s",
            num_cores=NC, num_subcores=NS),
        scratch_types=[
            pltpu.VMEM((16, 128), jnp.float32),
            pltpu.VMEM((16, 128), jnp.float32),
            pltpu.VMEM((16, 128), jnp.float32),
            pltpu.VMEM((16, 128), jnp.float32),
            pltpu.SemaphoreType.DMA,
            pltpu.SemaphoreType.DMA,
            pltpu.SemaphoreType.DMA,
            pltpu.SemaphoreType.DMA,
        ],
        compiler_params=pltpu.CompilerParams(
            use_tc_tiling_on_sc=True, needs_layout_passes=False),
    )


def _sc_body(idx_hbm, idxhi_hbm, emb_hbm, one16_hbm, feat_hbm, ones_hbm,
             idx_v, idxhi_v, rows_v, onesrows_v, ones_buf, sem_e, sem_o):
    wid = lax.axis_index("s") * NC + lax.axis_index("c")
    row0 = wid * ROWS_PER_W
    emb2d = emb_hbm
    one2d = one16_hbm
    pltpu.sync_copy(idx_hbm.at[pl.ds(row0, ROWS_PER_W)], idx_v)
    pltpu.sync_copy(idxhi_hbm.at[pl.ds(row0, ROWS_PER_W)], idxhi_v)
    for c in range(NCHUNK):
        handles = []
        for j in range(CHUNK):
            r = c * CHUNK + j
            handles.append(pltpu.async_copy(
                emb2d.at[idx_v.at[r]],
                rows_v.at[pl.ds(j * IDX_COLS, IDX_COLS)], sem_e))
            handles.append(pltpu.async_copy(
                one2d.at[idxhi_v.at[r]],
                onesrows_v.at[pl.ds(j * IDX_COLS, IDX_COLS)], sem_o))
        for h in handles:
            h.wait()
        base = (row0 + c * CHUNK) * IDX_COLS
        pltpu.sync_copy(rows_v, feat_hbm.at[pl.ds(base, CHUNK_IDX)])

        def lane_select(j, _):
            row = c * CHUNK + j
            for k in range(IDX_COLS // 16):
                iv = idx_v[row, pl.ds(k * 16, 16)]
                lane = jnp.bitwise_and(iv, 15)
                off = j * IDX_COLS + k * 16
                rid = lax.iota(jnp.int32, 16) + off
                ones_buf[pl.ds(off, 16)] = plsc.load_gather(
                    onesrows_v, [rid, lane])
            return 0

        lax.fori_loop(0, CHUNK, lane_select, 0)
        pltpu.sync_copy(ones_buf, ones_hbm.at[pl.ds(base, CHUNK_IDX)])


@functools.cache
def _sc_gather():
    return pl.kernel(
        _sc_body,
        out_type=(
            jax.ShapeDtypeStruct((TOT, EMB_DIM), jnp.float32),
            jax.ShapeDtypeStruct((TOT,), jnp.float32),
        ),
        mesh=plsc.VectorSubcoreMesh(
            core_axis_name="c", subcore_axis_name="s",
            num_cores=NC, num_subcores=NS),
        scratch_types=[
            pltpu.VMEM((ROWS_PER_W, IDX_COLS), jnp.int32),
            pltpu.VMEM((ROWS_PER_W, IDX_COLS), jnp.int32),
            pltpu.VMEM((CHUNK_IDX, EMB_DIM), jnp.float32),
            pltpu.VMEM((CHUNK_IDX, EMB_DIM), jnp.float32),
            pltpu.VMEM((CHUNK_IDX,), jnp.float32),
            pltpu.SemaphoreType.DMA,
            pltpu.SemaphoreType.DMA,
        ],
        compiler_params=pltpu.CompilerParams(
            use_tc_tiling_on_sc=False, needs_layout_passes=False),
    )


R = 256
GRID = BATCH // R


def _tc_body(feat_ref, ones_ref, d_ref, dw_ref, w1_ref, S_ref, M_ref,
             W0a_ref, b0_ref, W1_ref, b1_ref, W2_ref, b2_ref, W3_ref, b3_ref,
             out_ref):
    x = feat_ref[...]
    d = d_ref[...]
    dw = dw_ref[...]
    S = S_ref[...]

    dot = functools.partial(jnp.dot, preferred_element_type=jnp.float32)

    y1 = jnp.sum(ones_ref[...], axis=1, keepdims=True) + dot(d, w1_ref[...])

    s = dot(x, S) + dot(d, dw)
    sq = dot(x * x, S) + dot(d * d, dw * dw)
    y2 = 0.5 * jnp.sum(s * s - sq, axis=1, keepdims=True)

    h = jnp.maximum(dot(x, W0a_ref[...]) + dot(d, M_ref[...]) + b0_ref[...], 0.0)
    h = jnp.maximum(dot(h, W1_ref[...]) + b1_ref[...], 0.0)
    h = jnp.maximum(dot(h, W2_ref[...]) + b2_ref[...], 0.0)
    yd = dot(h, W3_ref[...]) + b3_ref[...]

    out_ref[...] = jax.nn.sigmoid(y1 + y2 + yd)


def _tc_forward(feat, ones, dense, dw, w_one, S, M,
                W0a, b0, W1, b1, W2, b2, W3, b3):
    full = lambda shape: pl.BlockSpec(shape, lambda i: (0, 0))
    return pl.pallas_call(
        _tc_body,
        grid=(GRID,),
        in_specs=[
            pl.BlockSpec((R, SPARSE_FLAT), lambda i: (i, 0)),
            pl.BlockSpec((R, N_SPARSE), lambda i: (i, 0)),
            pl.BlockSpec((R, DENSE_DIM), lambda i: (i, 0)),
            full((DENSE_DIM, EMB_DIM)),
            full((DENSE_DIM, 1)),
            full((SPARSE_FLAT, EMB_DIM)),
            full((DENSE_DIM, 512)),
            full((SPARSE_FLAT, 512)),
            full((1, 512)),
            full((512, 256)),
            full((1, 256)),
            full((256, 128)),
            full((1, 128)),
            full((128, 1)),
            full((1, 1)),
        ],
        out_specs=pl.BlockSpec((R, 1), lambda i: (i, 0)),
        out_shape=jax.ShapeDtypeStruct((BATCH, 1), jnp.float32),
    )(feat, ones, dense, dw, w_one, S, M, W0a, b0, W1, b1, W2, b2, W3, b3)


def kernel(sparse_inputs, dense_inputs, emb_one, emb, dense_w_one, dense_w,
           W0, b0, W1, b1, W2, b2, W3, b3):
    idx = sparse_inputs.astype(jnp.int32).reshape(IDX_ROWS, IDX_COLS)
    tail128 = emb[NCB * 128:].reshape(TAIL * EMB_DIM // 128, 128)
    emb_lin = _sc_detile()(emb.T, tail128).reshape(VOCAB, EMB_DIM)
    feat_flat, ones_flat = _sc_gather()(
        idx, jnp.right_shift(idx, 4), emb_lin,
        emb_one.reshape(VOCAB // EMB_DIM, EMB_DIM))
    feat = feat_flat.reshape(BATCH, SPARSE_FLAT)
    ones = ones_flat.reshape(BATCH, N_SPARSE)

    dw = dense_w[0]
    W0a = W0[:SPARSE_FLAT]
    W0b = W0[SPARSE_FLAT:]
    M = jnp.einsum('dk,dkj->dj', dw, W0b.reshape(DENSE_DIM, EMB_DIM, 512))
    S = jnp.tile(jnp.eye(EMB_DIM, dtype=jnp.float32), (N_SPARSE, 1))

    return _tc_forward(
        feat, ones, dense_inputs, dw, dense_w_one.reshape(DENSE_DIM, 1), S, M,
        W0a, b0.reshape(1, -1), W1, b1.reshape(1, -1),
        W2, b2.reshape(1, -1), W3, b3.reshape(1, 1))
